# baseline (speedup 1.0000x reference)
import functools

import jax
import jax.numpy as jnp
import numpy as np
from jax import lax
from jax.experimental import pallas as pl
from jax.experimental.pallas import tpu as pltpu
from jax.experimental.pallas import tpu_sc as plsc

N_NODES = 10000
N_PAD = 10240
N_EDGES = 320000
D = 128
N_USERS = 5000
BATCH_N = 4096

NC = 2
NS = 16
NW = NC * NS
EPW = N_EDGES // NW
CHUNK = 64
NFULL = EPW // CHUNK
TAIL = EPW - NFULL * CHUNK
ROWS_PT = N_PAD // NS

BPW = BATCH_N // NW

_PERM = np.concatenate(
    [np.concatenate([np.arange(32 * v, 32 * v + 32, 2),
                     np.arange(32 * v + 1, 32 * v + 32, 2)])
     for v in range(D // 32)])
_PM_NP = np.zeros((D, D), np.float32)
_PM_NP[np.arange(D), _PERM] = 1.0

_mesh = plsc.VectorSubcoreMesh(core_axis_name="c", subcore_axis_name="s")

_GDN = lax.GatherDimensionNumbers(
    offset_dims=(), collapsed_slice_dims=(0,), start_index_map=(0,))


def _lane_perm(vec16, idx16):
    return lax.gather(vec16, idx16.reshape(16, 1).astype(jnp.int32), _GDN,
                      (1,), mode=lax.GatherScatterMode.PROMISE_IN_BOUNDS)


def _lane_bcast(vec16, l):
    return _lane_perm(vec16, jnp.full((16,), l, jnp.int32))


def _bf16_pair_to_f32(u):
    a = lax.bitcast_convert_type(lax.shift_left(u, 16), jnp.float32)
    b = lax.bitcast_convert_type(lax.bitwise_and(u, jnp.int32(-65536)),
                                 jnp.float32)
    return a, b


def _lane_sum(vec16):
    lane = lax.iota(jnp.int32, 16)
    for sh in (1, 2, 4, 8):
        vec16 = vec16 + _lane_perm(vec16, lane ^ sh)
    return vec16


@functools.partial(
    pl.kernel,
    out_type=(
        jax.ShapeDtypeStruct((N_PAD, D), jnp.float32),
        jax.ShapeDtypeStruct((N_PAD, D), jnp.float32),
    ),
    mesh=_mesh,
    compiler_params=pltpu.CompilerParams(use_tc_tiling_on_sc=False),
    scratch_types=[
        pltpu.VMEM_SHARED((N_PAD, D), jnp.float32),
        pltpu.VMEM((CHUNK,), jnp.int32),
        pltpu.VMEM((CHUNK,), jnp.int32),
        pltpu.VMEM((CHUNK,), jnp.int32),
        pltpu.VMEM((CHUNK,), jnp.int32),
        pltpu.VMEM((CHUNK,), jnp.int32),
        pltpu.VMEM((CHUNK,), jnp.int32),
        pltpu.VMEM((CHUNK,), jnp.float32),
        pltpu.VMEM((CHUNK,), jnp.float32),
        pltpu.VMEM((CHUNK,), jnp.float32),
        pltpu.VMEM((CHUNK, D // 2), jnp.int32),
        pltpu.VMEM((CHUNK, D // 2), jnp.int32),
        pltpu.VMEM((CHUNK, D // 2), jnp.int32),
        pltpu.VMEM((CHUNK, D), jnp.float32),
        pltpu.VMEM((CHUNK, D), jnp.float32),
        pltpu.VMEM((CHUNK, D), jnp.float32),
        pltpu.VMEM((TAIL,), jnp.int32),
        pltpu.VMEM((TAIL,), jnp.int32),
        pltpu.VMEM((TAIL,), jnp.float32),
        pltpu.VMEM((TAIL, D // 2), jnp.int32),
        pltpu.VMEM((TAIL, D), jnp.float32),
        pltpu.SemaphoreType.DMA,
        pltpu.SemaphoreType.DMA,
        pltpu.SemaphoreType.DMA,
        pltpu.SemaphoreType.DMA,
        pltpu.SemaphoreType.DMA,
        pltpu.SemaphoreType.DMA,
        pltpu.SemaphoreType.DMA,
        pltpu.SemaphoreType.DMA,
        pltpu.SemaphoreType.DMA,
        pltpu.SemaphoreType.DMA,
        pltpu.SemaphoreType.DMA,
        pltpu.SemaphoreType.DMA,
    ],
)
def _prop(h_hbm, src_hbm, dst_hbm, w_hbm, out0, out1,
          acc, src0, src1, src2, dst0, dst1, dst2, w0, w1, w2,
          rb0, rb1, rb2, rf0, rf1, rf2,
          src_t, dst_t, w_t, rb_t, rf_t,
          sf0, sf1, sf2, sd0, sd1, sd2, sg0, sg1, sg2, sw0, sw1, sw2):
    c = lax.axis_index("c")
    s = lax.axis_index("s")
    wid = c * NS + s
    ebase = wid * EPW

    srcs = (src0, src1, src2)
    dsts = (dst0, dst1, dst2)
    ws = (w0, w1, w2)
    rbs = (rb0, rb1, rb2)
    rfs = (rf0, rf1, rf2)
    sfs = (sf0, sf1, sf2)
    sds = (sd0, sd1, sd2)
    sgs = (sg0, sg1, sg2)
    sws = (sw0, sw1, sw2)

    def _fetch_srcw(ci, x):
        off = ebase + ci * CHUNK
        pltpu.async_copy(src_hbm.at[pl.ds(off, CHUNK)], srcs[x], sfs[x])
        pltpu.async_copy(w_hbm.at[pl.ds(off, CHUNK)], ws[x], sfs[x])

    def _fwait(x):
        pltpu.make_async_copy(src_hbm.at[pl.ds(0, CHUNK)], srcs[x],
                              sfs[x]).wait()
        pltpu.make_async_copy(w_hbm.at[pl.ds(0, CHUNK)], ws[x], sfs[x]).wait()

    def _fetch_dst(ci, x):
        pltpu.async_copy(
            dst_hbm.at[pl.ds(ebase + ci * CHUNK, CHUNK)], dsts[x], sds[x])

    def _dwait(x):
        pltpu.make_async_copy(dst_hbm.at[pl.ds(0, CHUNK)], dsts[x],
                              sds[x]).wait()

    def _gissue(x):
        pltpu.async_copy(h_hbm.at[srcs[x]], rbs[x], sgs[x])

    def _gwait(x):
        pltpu.make_async_copy(h_hbm.at[pl.ds(0, CHUNK)], rbs[x],
                              sgs[x]).wait()

    def _wissue(x):
        pltpu.async_copy(rfs[x], acc.at[dsts[x]], sws[x], add=True)

    def _wwait(x):
        pltpu.make_async_copy(rfs[x], acc.at[dsts[x]], sws[x]).wait()

    def _scale_unpack(wbuf, rb, rf, nedge):
        def _grp(g, _):
            w16 = wbuf[pl.ds(g * 16, 16)]
            for l in range(16):
                wb = _lane_bcast(w16, l)
                j = g * 16 + l
                for v in range(D // 32):
                    u16 = rb[j, pl.ds(16 * v, 16)]
                    a, b = _bf16_pair_to_f32(u16)
                    rf[j, pl.ds(32 * v, 16)] = a * wb
                    rf[j, pl.ds(32 * v + 16, 16)] = b * wb
            return 0

        lax.fori_loop(0, nedge // 16, _grp, 0)

    zvec = jnp.zeros((16,), jnp.float32)

    def _zrow(i, _):
        for v in range(D // 16):
            rf0[i, pl.ds(16 * v, 16)] = zvec
        return 0

    lax.fori_loop(0, CHUNK, _zrow, 0)
    for r in range(ROWS_PT // CHUNK):
        pltpu.sync_copy(rf0, acc.at[pl.ds(s * ROWS_PT + r * CHUNK, CHUNK)])
    plsc.subcore_barrier()

    def _step(i, x, first=False):
        z = (x + 2) % 3
        _gwait(x)
        _scale_unpack(ws[x], rbs[x], rfs[x], CHUNK)
        _dwait(x)
        _wissue(x)
        if not first:
            _wwait(z)

        @pl.when(i + 2 < NFULL)
        def _():
            _fetch_dst(i + 2, z)

        @pl.when(i + 3 < NFULL)
        def _():
            _fetch_srcw(i + 3, x)

        @pl.when(i + 2 < NFULL)
        def _():
            _fwait(z)
            _gissue(z)

    _fetch_srcw(0, 0)
    _fetch_srcw(1, 1)
    _fetch_srcw(2, 2)
    _fetch_dst(0, 0)
    _fetch_dst(1, 1)
    _fwait(0)
    _gissue(0)
    _fwait(1)
    _gissue(1)
    _step(0, 0, first=True)

    def _triple(t, _):
        i = 3 * t + 1
        _step(i, 1)
        _step(i + 1, 2)
        _step(i + 2, 0)
        return 0

    lax.fori_loop(0, (NFULL - 3) // 3, _triple, 0)
    _step(NFULL - 2, (NFULL - 2) % 3)
    _step(NFULL - 1, (NFULL - 1) % 3)
    _wwait((NFULL - 1) % 3)

    toff = ebase + NFULL * CHUNK
    pltpu.sync_copy(src_hbm.at[pl.ds(toff, TAIL)], src_t)
    pltpu.sync_copy(dst_hbm.at[pl.ds(toff, TAIL)], dst_t)
    pltpu.sync_copy(w_hbm.at[pl.ds(toff, TAIL)], w_t)
    pltpu.async_copy(h_hbm.at[src_t], rb_t, sg0).wait()
    w16 = w_t[pl.ds(0, 16)]
    for l in range(TAIL):
        wb = _lane_bcast(w16, l)
        for v in range(D // 32):
            u16 = rb_t[l, pl.ds(16 * v, 16)]
            a, b = _bf16_pair_to_f32(u16)
            rf_t[l, pl.ds(32 * v, 16)] = a * wb
            rf_t[l, pl.ds(32 * v + 16, 16)] = b * wb
    pltpu.sync_copy(rf_t, acc.at[dst_t], add=True)
    plsc.subcore_barrier()

    @pl.when(c == 0)
    def _():
        pltpu.sync_copy(acc.at[pl.ds(s * ROWS_PT, ROWS_PT)],
                        out0.at[pl.ds(s * ROWS_PT, ROWS_PT)])

    @pl.when(c == 1)
    def _():
        pltpu.sync_copy(acc.at[pl.ds(s * ROWS_PT, ROWS_PT)],
                        out1.at[pl.ds(s * ROWS_PT, ROWS_PT)])


def _combine_body(p0_ref, p1_ref, s_ref, pm_ref, hb_ref, so_ref):
    fx = jnp.dot(p0_ref[...] + p1_ref[...], pm_ref[...],
                 preferred_element_type=jnp.float32)
    hb_ref[...] = fx.astype(jnp.bfloat16)
    so_ref[...] = s_ref[...] + fx


_CROWS = 1024


def _combine(p0, p1, s_in, pm):
    spec = pl.BlockSpec((_CROWS, D), lambda i: (i, 0))
    pm_spec = pl.BlockSpec((D, D), lambda i: (0, 0))
    return pl.pallas_call(
        _combine_body,
        grid=(N_PAD // _CROWS,),
        in_specs=[spec, spec, spec, pm_spec],
        out_specs=[spec, spec],
        out_shape=[
            jax.ShapeDtypeStruct((N_PAD, D), jnp.bfloat16),
            jax.ShapeDtypeStruct((N_PAD, D), jnp.float32),
        ],
    )(p0, p1, s_in, pm)


@functools.partial(
    pl.kernel,
    out_type=jax.ShapeDtypeStruct((BATCH_N,), jnp.float32),
    mesh=_mesh,
    scratch_types=[
        pltpu.VMEM((BPW,), jnp.int32),
        pltpu.VMEM((BPW,), jnp.int32),
        pltpu.VMEM((BPW,), jnp.int32),
        pltpu.VMEM((BPW, D), jnp.float32),
        pltpu.VMEM((BPW, D), jnp.float32),
        pltpu.VMEM((BPW,), jnp.float32),
        pltpu.SemaphoreType.DMA,
    ],
)
def _gamma(sum_hbm, users_hbm, items_hbm, out_hbm,
           u_v, it_v, ii_v, ur_v, ir_v, out_v, sem):
    wid = lax.axis_index("c") * NS + lax.axis_index("s")
    off = wid * BPW
    pltpu.sync_copy(users_hbm.at[pl.ds(off, BPW)], u_v)
    pltpu.sync_copy(items_hbm.at[pl.ds(off, BPW)], it_v)

    def _shift(i, _):
        sl = pl.ds(i * 16, 16)
        ii_v[sl] = it_v[sl] + N_USERS
        return 0

    lax.fori_loop(0, BPW // 16, _shift, 0)
    pltpu.async_copy(sum_hbm.at[u_v], ur_v, sem).wait()
    pltpu.async_copy(sum_hbm.at[ii_v], ir_v, sem).wait()

    lane = lax.iota(jnp.int32, 16)

    def _group(g, _):
        def _one(l, accv):
            b = g * 16 + l
            acc = jnp.zeros((16,), jnp.float32)
            for v in range(D // 16):
                sl = pl.ds(16 * v, 16)
                acc = acc + ur_v[b, sl] * ir_v[b, sl]
            gvec = _lane_sum(acc) * (1.0 / 16.0)
            return jnp.where(lane == l, gvec, accv)

        vec = lax.fori_loop(0, 16, _one, jnp.zeros((16,), jnp.float32))
        out_v[pl.ds(g * 16, 16)] = vec
        return 0

    lax.fori_loop(0, BPW // 16, _group, 0)
    pltpu.sync_copy(out_v, out_hbm.at[pl.ds(off, BPW)])


def kernel(user_emb, item_emb, edge_index, edge_weight, users, items):
    all_emb = jnp.concatenate([user_emb, item_emb], axis=0)
    src = edge_index[0].astype(jnp.int32)
    dst = edge_index[1].astype(jnp.int32)
    users = users.astype(jnp.int32)
    items = items.astype(jnp.int32)
    pm = jnp.asarray(_PM_NP)

    pad = jnp.zeros((N_PAD - N_NODES, D), jnp.float32)
    h_true = jnp.concatenate([all_emb, pad], axis=0)
    ssum = h_true

    def _pack_rows(hbf):
        return lax.bitcast_convert_type(
            hbf.reshape(N_PAD, D // 2, 2), jnp.int32)

    h_pk = _pack_rows(h_true.astype(jnp.bfloat16))
    for _ in range(3):
        p0, p1 = _prop(h_pk, src, dst, edge_weight)
        h_bf, ssum = _combine(p0, p1, ssum, pm)
        h_pk = _pack_rows(h_bf)
    return _gamma(ssum, users, items)


# speedup vs baseline: 2.1788x; 2.1788x over previous
import functools

import jax
import jax.numpy as jnp
from jax import lax
from jax.experimental import pallas as pl
from jax.experimental.pallas import tpu as pltpu
from jax.experimental.pallas import tpu_sc as plsc

N_NODES = 10000
N_PAD = 10240
N_EDGES = 320000
D = 128
N_USERS = 5000
BATCH_N = 4096

NC = 2
NS = 16
NW = NC * NS
EPW = N_EDGES // NW
CHUNK = 112
NFULL = EPW // CHUNK
TAIL = EPW - NFULL * CHUNK
ROWS_PT = N_PAD // NS
ZCH = 128

BPW = BATCH_N // NW

_mesh = plsc.VectorSubcoreMesh(core_axis_name="c", subcore_axis_name="s")

_GDN = lax.GatherDimensionNumbers(
    offset_dims=(), collapsed_slice_dims=(0,), start_index_map=(0,))


def _lane_perm(vec16, idx16):
    return lax.gather(vec16, idx16.reshape(16, 1).astype(jnp.int32), _GDN,
                      (1,), mode=lax.GatherScatterMode.PROMISE_IN_BOUNDS)


def _lane_bcast(vec16, l):
    return _lane_perm(vec16, jnp.full((16,), l, jnp.int32))


def _lane_sum(vec16):
    lane = lax.iota(jnp.int32, 16)
    for sh in (1, 2, 4, 8):
        vec16 = vec16 + _lane_perm(vec16, lane ^ sh)
    return vec16


@functools.partial(
    pl.kernel,
    out_type=(
        jax.ShapeDtypeStruct((N_PAD, D), jnp.float32),
        jax.ShapeDtypeStruct((N_PAD, D), jnp.float32),
    ),
    mesh=_mesh,
    scratch_types=[
        pltpu.VMEM_SHARED((N_PAD, D), jnp.float32),
        pltpu.VMEM((CHUNK,), jnp.int32),
        pltpu.VMEM((CHUNK,), jnp.int32),
        pltpu.VMEM((CHUNK,), jnp.int32),
        pltpu.VMEM((CHUNK,), jnp.int32),
        pltpu.VMEM((CHUNK,), jnp.int32),
        pltpu.VMEM((CHUNK,), jnp.int32),
        pltpu.VMEM((CHUNK,), jnp.float32),
        pltpu.VMEM((CHUNK,), jnp.float32),
        pltpu.VMEM((CHUNK,), jnp.float32),
        pltpu.VMEM((CHUNK, D), jnp.float32),
        pltpu.VMEM((CHUNK, D), jnp.float32),
        pltpu.VMEM((CHUNK, D), jnp.float32),
        pltpu.VMEM((TAIL,), jnp.int32),
        pltpu.VMEM((TAIL,), jnp.int32),
        pltpu.VMEM((TAIL,), jnp.float32),
        pltpu.VMEM((TAIL, D), jnp.float32),
        pltpu.SemaphoreType.DMA,
        pltpu.SemaphoreType.DMA,
        pltpu.SemaphoreType.DMA,
        pltpu.SemaphoreType.DMA,
        pltpu.SemaphoreType.DMA,
        pltpu.SemaphoreType.DMA,
        pltpu.SemaphoreType.DMA,
        pltpu.SemaphoreType.DMA,
        pltpu.SemaphoreType.DMA,
        pltpu.SemaphoreType.DMA,
        pltpu.SemaphoreType.DMA,
        pltpu.SemaphoreType.DMA,
    ],
)
def _prop(h_hbm, src_hbm, dst_hbm, w_hbm, out0, out1,
          acc, src0, src1, src2, dst0, dst1, dst2, w0, w1, w2,
          rw0, rw1, rw2, src_t, dst_t, w_t, rw_t,
          sf0, sf1, sf2, sd0, sd1, sd2, sg0, sg1, sg2, sw0, sw1, sw2):
    c = lax.axis_index("c")
    s = lax.axis_index("s")
    wid = c * NS + s
    ebase = wid * EPW

    srcs = (src0, src1, src2)
    dsts = (dst0, dst1, dst2)
    ws = (w0, w1, w2)
    rows = (rw0, rw1, rw2)
    sfs = (sf0, sf1, sf2)
    sds = (sd0, sd1, sd2)
    sgs = (sg0, sg1, sg2)
    sws = (sw0, sw1, sw2)

    def _fetch_srcw(ci, x):
        off = ebase + ci * CHUNK
        pltpu.async_copy(src_hbm.at[pl.ds(off, CHUNK)], srcs[x], sfs[x])
        pltpu.async_copy(w_hbm.at[pl.ds(off, CHUNK)], ws[x], sfs[x])

    def _fwait(x):
        pltpu.make_async_copy(src_hbm.at[pl.ds(0, CHUNK)], srcs[x],
                              sfs[x]).wait()
        pltpu.make_async_copy(w_hbm.at[pl.ds(0, CHUNK)], ws[x], sfs[x]).wait()

    def _fetch_dst(ci, x):
        pltpu.async_copy(
            dst_hbm.at[pl.ds(ebase + ci * CHUNK, CHUNK)], dsts[x], sds[x])

    def _dwait(x):
        pltpu.make_async_copy(dst_hbm.at[pl.ds(0, CHUNK)], dsts[x],
                              sds[x]).wait()

    def _gissue(x):
        pltpu.async_copy(h_hbm.at[srcs[x]], rows[x], sgs[x])

    def _gwait(x):
        pltpu.make_async_copy(h_hbm.at[pl.ds(0, CHUNK)], rows[x],
                              sgs[x]).wait()

    def _wissue(x):
        pltpu.async_copy(rows[x], acc.at[dsts[x]], sws[x], add=True)

    def _wwait(x):
        pltpu.make_async_copy(rows[x], acc.at[dsts[x]], sws[x]).wait()

    def _scale(wbuf, rbuf, nedge):
        def _grp(g, _):
            w16 = wbuf[pl.ds(g * 16, 16)]
            for l in range(16):
                wb = _lane_bcast(w16, l)
                j = g * 16 + l
                for v in range(D // 16):
                    rsl = pl.ds(16 * v, 16)
                    rbuf[j, rsl] = rbuf[j, rsl] * wb
            return 0

        lax.fori_loop(0, nedge // 16, _grp, 0)

    zvec = jnp.zeros((16,), jnp.float32)

    def _zrow(i, _):
        for v in range(D // 16):
            rw0[i, pl.ds(16 * v, 16)] = zvec
        return 0

    lax.fori_loop(0, CHUNK, _zrow, 0)
    for r in range(ROWS_PT // CHUNK):
        pltpu.sync_copy(rw0.at[pl.ds(0, CHUNK)],
                        acc.at[pl.ds(s * ROWS_PT + r * CHUNK, CHUNK)])
    zrem = ROWS_PT - (ROWS_PT // CHUNK) * CHUNK
    pltpu.sync_copy(rw0.at[pl.ds(0, zrem)],
                    acc.at[pl.ds(s * ROWS_PT + ROWS_PT - zrem, zrem)])
    plsc.subcore_barrier()

    def _step(i, x, first=False):
        z = (x + 2) % 3
        _gwait(x)
        _scale(ws[x], rows[x], CHUNK)
        _dwait(x)
        _wissue(x)
        if not first:
            _wwait(z)

        @pl.when(i + 2 < NFULL)
        def _():
            _fetch_dst(i + 2, z)

        @pl.when(i + 3 < NFULL)
        def _():
            _fetch_srcw(i + 3, x)

        @pl.when(i + 2 < NFULL)
        def _():
            _fwait(z)
            _gissue(z)

    _fetch_srcw(0, 0)
    _fetch_srcw(1, 1)
    _fetch_srcw(2, 2)
    _fetch_dst(0, 0)
    _fetch_dst(1, 1)
    _fwait(0)
    _gissue(0)
    _fwait(1)
    _gissue(1)
    _step(0, 0, first=True)

    def _triple(t, _):
        i = 3 * t + 1
        _step(i, 1)
        _step(i + 1, 2)
        _step(i + 2, 0)
        return 0

    ntrip = (NFULL - 1) // 3
    lax.fori_loop(0, ntrip, _triple, 0)
    for i in range(3 * ntrip + 1, NFULL):
        _step(i, i % 3)
    _wwait((NFULL - 1) % 3)

    toff = ebase + NFULL * CHUNK
    pltpu.sync_copy(src_hbm.at[pl.ds(toff, TAIL)], src_t)
    pltpu.sync_copy(dst_hbm.at[pl.ds(toff, TAIL)], dst_t)
    pltpu.sync_copy(w_hbm.at[pl.ds(toff, TAIL)], w_t)
    pltpu.async_copy(h_hbm.at[src_t], rw_t, sg0).wait()
    _scale(w_t, rw_t, TAIL)
    pltpu.sync_copy(rw_t, acc.at[dst_t], add=True)
    plsc.subcore_barrier()

    @pl.when(c == 0)
    def _():
        pltpu.sync_copy(acc.at[pl.ds(s * ROWS_PT, ROWS_PT)],
                        out0.at[pl.ds(s * ROWS_PT, ROWS_PT)])

    @pl.when(c == 1)
    def _():
        pltpu.sync_copy(acc.at[pl.ds(s * ROWS_PT, ROWS_PT)],
                        out1.at[pl.ds(s * ROWS_PT, ROWS_PT)])


def _combine_body(p0_ref, p1_ref, s_ref, h_ref, so_ref):
    h = p0_ref[...] + p1_ref[...]
    h_ref[...] = h
    so_ref[...] = s_ref[...] + h


_CROWS = 1024


def _combine(p0, p1, s_in):
    spec = pl.BlockSpec((_CROWS, D), lambda i: (i, 0))
    return pl.pallas_call(
        _combine_body,
        grid=(N_PAD // _CROWS,),
        in_specs=[spec, spec, spec],
        out_specs=[spec, spec],
        out_shape=[
            jax.ShapeDtypeStruct((N_PAD, D), jnp.float32),
            jax.ShapeDtypeStruct((N_PAD, D), jnp.float32),
        ],
    )(p0, p1, s_in)


@functools.partial(
    pl.kernel,
    out_type=jax.ShapeDtypeStruct((BATCH_N,), jnp.float32),
    mesh=_mesh,
    scratch_types=[
        pltpu.VMEM((BPW,), jnp.int32),
        pltpu.VMEM((BPW,), jnp.int32),
        pltpu.VMEM((BPW,), jnp.int32),
        pltpu.VMEM((BPW, D), jnp.float32),
        pltpu.VMEM((BPW, D), jnp.float32),
        pltpu.VMEM((BPW, D), jnp.float32),
        pltpu.VMEM((BPW, D), jnp.float32),
        pltpu.VMEM((BPW, D), jnp.float32),
        pltpu.VMEM((BPW, D), jnp.float32),
        pltpu.VMEM((BPW,), jnp.float32),
        pltpu.SemaphoreType.DMA,
    ],
)
def _gamma3(s2_hbm, p0_hbm, p1_hbm, users_hbm, items_hbm, out_hbm,
            u_v, it_v, ii_v, us_v, up0_v, up1_v, is_v, ip0_v, ip1_v,
            out_v, sem):
    wid = lax.axis_index("c") * NS + lax.axis_index("s")
    off = wid * BPW
    pltpu.sync_copy(users_hbm.at[pl.ds(off, BPW)], u_v)
    pltpu.sync_copy(items_hbm.at[pl.ds(off, BPW)], it_v)

    def _shift(i, _):
        sl = pl.ds(i * 16, 16)
        ii_v[sl] = it_v[sl] + N_USERS
        return 0

    lax.fori_loop(0, BPW // 16, _shift, 0)
    pltpu.async_copy(s2_hbm.at[u_v], us_v, sem)
    pltpu.async_copy(p0_hbm.at[u_v], up0_v, sem)
    pltpu.async_copy(p1_hbm.at[u_v], up1_v, sem)
    pltpu.async_copy(s2_hbm.at[ii_v], is_v, sem)
    pltpu.async_copy(p0_hbm.at[ii_v], ip0_v, sem)
    pltpu.async_copy(p1_hbm.at[ii_v], ip1_v, sem)
    for buf, hbm in ((us_v, s2_hbm), (up0_v, p0_hbm), (up1_v, p1_hbm),
                     (is_v, s2_hbm), (ip0_v, p0_hbm), (ip1_v, p1_hbm)):
        pltpu.make_async_copy(hbm.at[pl.ds(0, BPW)], buf, sem).wait()

    lane = lax.iota(jnp.int32, 16)

    def _group(g, _):
        def _one(l, accv):
            b = g * 16 + l
            acc = jnp.zeros((16,), jnp.float32)
            for v in range(D // 16):
                sl = pl.ds(16 * v, 16)
                urow = us_v[b, sl] + up0_v[b, sl] + up1_v[b, sl]
                irow = is_v[b, sl] + ip0_v[b, sl] + ip1_v[b, sl]
                acc = acc + urow * irow
            gvec = _lane_sum(acc) * (1.0 / 16.0)
            return jnp.where(lane == l, gvec, accv)

        vec = lax.fori_loop(0, 16, _one, jnp.zeros((16,), jnp.float32))
        out_v[pl.ds(g * 16, 16)] = vec
        return 0

    lax.fori_loop(0, BPW // 16, _group, 0)
    pltpu.sync_copy(out_v, out_hbm.at[pl.ds(off, BPW)])


def kernel(user_emb, item_emb, edge_index, edge_weight, users, items):
    all_emb = jnp.concatenate([user_emb, item_emb], axis=0)
    src = edge_index[0].astype(jnp.int32)
    dst = edge_index[1].astype(jnp.int32)
    users = users.astype(jnp.int32)
    items = items.astype(jnp.int32)

    pad = jnp.zeros((N_PAD - N_NODES, D), jnp.float32)
    h = jnp.concatenate([all_emb, pad], axis=0)
    ssum = h
    for _ in range(2):
        p0, p1 = _prop(h, src, dst, edge_weight)
        h, ssum = _combine(p0, p1, ssum)
    p0, p1 = _prop(h, src, dst, edge_weight)
    return _gamma3(ssum, p0, p1, users, items)
